# Initial kernel scaffold; baseline (speedup 1.0000x reference)
import functools

import jax
import jax.numpy as jnp
from jax import lax
from jax.experimental import pallas as pl
from jax.experimental.pallas import tpu as pltpu
from jax.experimental.pallas import tpu_sc as plsc

N = 100000
NPAD = 100352
E = 6400000
NC = 2
NS = 16
NW = NC * NS
EPT = E // NW
CHUNK = 2000
NCHUNK = EPT // CHUNK
SL = NPAD // NS

_mesh = plsc.VectorSubcoreMesh(core_axis_name="c", subcore_axis_name="s")
_f32 = jnp.float32


def _fill(ref, n, value):
    def body(i, carry):
        ref[pl.ds(i * 16, 16)] = jnp.full((16,), value, _f32)
        return carry
    lax.fori_loop(0, n // 16, body, 0)


@functools.partial(
    pl.kernel,
    out_type=(
        jax.ShapeDtypeStruct((NC, NPAD), _f32),
        jax.ShapeDtypeStruct((NC, NPAD), _f32),
    ),
    mesh=_mesh,
    scratch_types=[
        pltpu.VMEM_SHARED((NPAD,), _f32),
        pltpu.VMEM_SHARED((NPAD,), _f32),
        pltpu.VMEM((CHUNK,), jnp.int32),
        pltpu.VMEM((CHUNK,), jnp.int32),
        pltpu.VMEM((CHUNK,), jnp.int32),
        pltpu.VMEM((CHUNK,), jnp.int32),
        pltpu.VMEM((CHUNK,), _f32),
        pltpu.VMEM((SL,), _f32),
        pltpu.SemaphoreType.DMA,
        pltpu.SemaphoreType.DMA,
    ],
)
def _sc_degrees(ei, degs, degd, ds_s, dd_s, srcA, dstA, srcB, dstB, ones_v,
                zb, sem_i, sem_s):
    c = lax.axis_index("c")
    s = lax.axis_index("s")
    off = s * SL
    _fill(zb, SL, 0.0)
    _fill(ones_v, CHUNK, 1.0)
    pltpu.sync_copy(zb, ds_s.at[pl.ds(off, SL)])
    pltpu.sync_copy(zb, dd_s.at[pl.ds(off, SL)])
    plsc.subcore_barrier()

    base = (c * NS + s) * EPT
    bufs = ((srcA, dstA), (srcB, dstB))
    pltpu.async_copy(ei.at[0, pl.ds(base, CHUNK)], srcA, sem_i)
    pltpu.async_copy(ei.at[1, pl.ds(base, CHUNK)], dstA, sem_i)

    def chunk(k, cur, other):
        srcb, dstb = cur
        o = base + k * CHUNK
        pltpu.make_async_copy(ei.at[0, pl.ds(o, CHUNK)], srcb, sem_i).wait()
        pltpu.make_async_copy(ei.at[1, pl.ds(o, CHUNK)], dstb, sem_i).wait()

        @pl.when(k >= 1)
        def _():
            pltpu.make_async_copy(ones_v, ds_s.at[other[0]], sem_s).wait()
            pltpu.make_async_copy(ones_v, dd_s.at[other[1]], sem_s).wait()

        @pl.when(k < NCHUNK - 1)
        def _():
            o2 = base + (k + 1) * CHUNK
            pltpu.async_copy(ei.at[0, pl.ds(o2, CHUNK)], other[0], sem_i)
            pltpu.async_copy(ei.at[1, pl.ds(o2, CHUNK)], other[1], sem_i)

        pltpu.async_copy(ones_v, ds_s.at[srcb], sem_s, add=True)
        pltpu.async_copy(ones_v, dd_s.at[dstb], sem_s, add=True)

    def body(i, carry):
        chunk(2 * i, bufs[0], bufs[1])
        chunk(2 * i + 1, bufs[1], bufs[0])
        return carry
    lax.fori_loop(0, NCHUNK // 2, body, 0)
    pltpu.make_async_copy(ones_v, ds_s.at[srcB], sem_s).wait()
    pltpu.make_async_copy(ones_v, dd_s.at[dstB], sem_s).wait()

    plsc.subcore_barrier()
    pltpu.sync_copy(ds_s.at[pl.ds(off, SL)], degs.at[c, pl.ds(off, SL)])
    pltpu.sync_copy(dd_s.at[pl.ds(off, SL)], degd.at[c, pl.ds(off, SL)])


@functools.partial(
    pl.kernel,
    out_type=jax.ShapeDtypeStruct((NC, 2, NPAD), _f32),
    mesh=_mesh,
    scratch_types=[
        pltpu.VMEM_SHARED((NPAD,), _f32),
        pltpu.VMEM_SHARED((NPAD,), _f32),
        pltpu.VMEM_SHARED((NPAD,), _f32),
        pltpu.VMEM_SHARED((NPAD,), _f32),
        pltpu.VMEM((CHUNK,), jnp.int32),
        pltpu.VMEM((CHUNK,), jnp.int32),
        pltpu.VMEM((CHUNK,), jnp.int32),
        pltpu.VMEM((CHUNK,), jnp.int32),
        pltpu.VMEM((CHUNK,), _f32),
        pltpu.VMEM((CHUNK,), _f32),
        pltpu.VMEM((CHUNK,), _f32),
        pltpu.VMEM((CHUNK,), _f32),
        pltpu.VMEM((SL,), _f32),
        pltpu.SemaphoreType.DMA,
        pltpu.SemaphoreType.DMA,
        pltpu.SemaphoreType.DMA,
    ],
)
def _sc_pass(ei, tbl, out, t0_s, t1_s, a0_s, a1_s, srcA, dstA, srcB, dstB,
             v0A, v1A, v0B, v1B, zb, sem_i, sem_g, sem_s):
    c = lax.axis_index("c")
    s = lax.axis_index("s")
    off = s * SL
    _fill(zb, SL, 0.0)
    pltpu.sync_copy(tbl.at[0, pl.ds(off, SL)], t0_s.at[pl.ds(off, SL)])
    pltpu.sync_copy(tbl.at[1, pl.ds(off, SL)], t1_s.at[pl.ds(off, SL)])
    pltpu.sync_copy(zb, a0_s.at[pl.ds(off, SL)])
    pltpu.sync_copy(zb, a1_s.at[pl.ds(off, SL)])
    plsc.subcore_barrier()

    base = (c * NS + s) * EPT
    bufs = ((srcA, dstA, v0A, v1A), (srcB, dstB, v0B, v1B))
    pltpu.async_copy(ei.at[0, pl.ds(base, CHUNK)], srcA, sem_i)
    pltpu.async_copy(ei.at[1, pl.ds(base, CHUNK)], dstA, sem_i)

    def chunk(k, cur, other):
        srcb, dstb, v0, v1 = cur
        o = base + k * CHUNK
        pltpu.make_async_copy(ei.at[0, pl.ds(o, CHUNK)], srcb, sem_i).wait()
        pltpu.make_async_copy(ei.at[1, pl.ds(o, CHUNK)], dstb, sem_i).wait()

        g0 = pltpu.async_copy(t0_s.at[srcb], v0, sem_g)
        g1 = pltpu.async_copy(t1_s.at[srcb], v1, sem_g)

        @pl.when(k >= 1)
        def _():
            pltpu.make_async_copy(other[2], a0_s.at[other[1]], sem_s).wait()
            pltpu.make_async_copy(other[3], a1_s.at[other[1]], sem_s).wait()

        @pl.when(k < NCHUNK - 1)
        def _():
            o2 = base + (k + 1) * CHUNK
            pltpu.async_copy(ei.at[0, pl.ds(o2, CHUNK)], other[0], sem_i)
            pltpu.async_copy(ei.at[1, pl.ds(o2, CHUNK)], other[1], sem_i)

        g0.wait()
        g1.wait()
        pltpu.async_copy(v0, a0_s.at[dstb], sem_s, add=True)
        pltpu.async_copy(v1, a1_s.at[dstb], sem_s, add=True)

    def body(i, carry):
        chunk(2 * i, bufs[0], bufs[1])
        chunk(2 * i + 1, bufs[1], bufs[0])
        return carry
    lax.fori_loop(0, NCHUNK // 2, body, 0)
    pltpu.make_async_copy(v0B, a0_s.at[dstB], sem_s).wait()
    pltpu.make_async_copy(v1B, a1_s.at[dstB], sem_s).wait()

    plsc.subcore_barrier()
    pltpu.sync_copy(a0_s.at[pl.ds(off, SL)], out.at[c, 0, pl.ds(off, SL)])
    pltpu.sync_copy(a1_s.at[pl.ds(off, SL)], out.at[c, 1, pl.ds(off, SL)])


def _tc_prep_body(degs_ref, degd_ref, feat_ref, t2_ref, ns_ref, nd_ref):
    ds = degs_ref[0:1, :] + degs_ref[1:2, :]
    dd = degd_ref[0:1, :] + degd_ref[1:2, :]
    ns = lax.rsqrt(jnp.maximum(ds, 1.0))
    nd = lax.rsqrt(jnp.maximum(dd, 1.0))
    ns_ref[...] = ns
    nd_ref[...] = nd
    t2_ref[...] = feat_ref[...] * ns


def _tc_dense_body(acc_ref, ns_ref, nd_ref, w2t_ref, b2_ref, w3t_ref, y3_ref):
    a = (acc_ref[0] + acc_ref[1]) * nd_ref[...]
    x2 = jnp.dot(w2t_ref[...], a, preferred_element_type=_f32) + b2_ref[...]
    x2 = jnp.maximum(x2, 0.0)
    y3 = jnp.dot(w3t_ref[...], x2, preferred_element_type=_f32)
    y3_ref[...] = y3 * ns_ref[...]


def _tc_final_body(acc_ref, nd_ref, b3_ref, out_ref):
    out_ref[...] = (acc_ref[0] + acc_ref[1]) * nd_ref[...] + b3_ref[...]


def kernel(features, edge_index, theta_w, theta_b, phi_w, phi_b, w2, b2, w3, b3):
    featT = jnp.pad(features.T, ((0, 0), (0, NPAD - N)))

    degs, degd = _sc_degrees(edge_index)

    t2, ns, nd = pl.pallas_call(
        _tc_prep_body,
        out_shape=(
            jax.ShapeDtypeStruct((2, NPAD), _f32),
            jax.ShapeDtypeStruct((1, NPAD), _f32),
            jax.ShapeDtypeStruct((1, NPAD), _f32),
        ),
    )(degs, degd, featT)

    acc2 = _sc_pass(edge_index, t2)

    y3 = pl.pallas_call(
        _tc_dense_body,
        out_shape=jax.ShapeDtypeStruct((2, NPAD), _f32),
    )(acc2, ns, nd, w2.T, b2.reshape(8, 1), w3.T)

    acc3 = _sc_pass(edge_index, y3)

    outT = pl.pallas_call(
        _tc_final_body,
        out_shape=jax.ShapeDtypeStruct((2, NPAD), _f32),
    )(acc3, nd, b3.reshape(2, 1))

    return outT[:, :N].T


# speedup vs baseline: 114.2571x; 114.2571x over previous
import functools

import jax
import jax.numpy as jnp
from jax import lax
from jax.experimental import pallas as pl
from jax.experimental.pallas import tpu as pltpu
from jax.experimental.pallas import tpu_sc as plsc

N = 100000
NPAD = 100352
E = 6400000
NC = 2
NS = 16
NW = NC * NS
EPT = E // NW
CHUNK = 2000
NCHUNK = EPT // CHUNK
SL = NPAD // NS

_mesh = plsc.VectorSubcoreMesh(core_axis_name="c", subcore_axis_name="s")
_f32 = jnp.float32


def _fill(ref, n, value):
    def body(i, carry):
        ref[pl.ds(i * 16, 16)] = jnp.full((16,), value, _f32)
        return carry
    lax.fori_loop(0, n // 16, body, 0)


@functools.partial(
    pl.kernel,
    out_type=(
        jax.ShapeDtypeStruct((NC, NPAD), _f32),
        jax.ShapeDtypeStruct((NC, NPAD), _f32),
    ),
    mesh=_mesh,
    scratch_types=[
        pltpu.VMEM_SHARED((NPAD,), _f32),
        pltpu.VMEM_SHARED((NPAD,), _f32),
        pltpu.VMEM((CHUNK,), jnp.int32),
        pltpu.VMEM((CHUNK,), jnp.int32),
        pltpu.VMEM((CHUNK,), jnp.int32),
        pltpu.VMEM((CHUNK,), jnp.int32),
        pltpu.VMEM((CHUNK,), _f32),
        pltpu.VMEM((SL,), _f32),
        pltpu.SemaphoreType.DMA,
        pltpu.SemaphoreType.DMA,
    ],
)
def _sc_degrees(src_h, dst_h, degs, degd, ds_s, dd_s, srcA, dstA, srcB, dstB, ones_v,
                zb, sem_i, sem_s):
    c = lax.axis_index("c")
    s = lax.axis_index("s")
    off = s * SL
    _fill(zb, SL, 0.0)
    _fill(ones_v, CHUNK, 1.0)
    pltpu.sync_copy(zb, ds_s.at[pl.ds(off, SL)])
    pltpu.sync_copy(zb, dd_s.at[pl.ds(off, SL)])
    plsc.subcore_barrier()

    base = (c * NS + s) * EPT
    bufs = ((srcA, dstA), (srcB, dstB))
    pltpu.async_copy(src_h.at[pl.ds(base, CHUNK)], srcA, sem_i)
    pltpu.async_copy(dst_h.at[pl.ds(base, CHUNK)], dstA, sem_i)

    def chunk(k, cur, other):
        srcb, dstb = cur
        o = base + k * CHUNK
        pltpu.make_async_copy(src_h.at[pl.ds(o, CHUNK)], srcb, sem_i).wait()
        pltpu.make_async_copy(dst_h.at[pl.ds(o, CHUNK)], dstb, sem_i).wait()

        @pl.when(k >= 1)
        def _():
            pltpu.make_async_copy(ones_v, ds_s.at[other[0]], sem_s).wait()
            pltpu.make_async_copy(ones_v, dd_s.at[other[1]], sem_s).wait()

        @pl.when(k < NCHUNK - 1)
        def _():
            o2 = base + (k + 1) * CHUNK
            pltpu.async_copy(src_h.at[pl.ds(o2, CHUNK)], other[0], sem_i)
            pltpu.async_copy(dst_h.at[pl.ds(o2, CHUNK)], other[1], sem_i)

        pltpu.async_copy(ones_v, ds_s.at[srcb], sem_s, add=True)
        pltpu.async_copy(ones_v, dd_s.at[dstb], sem_s, add=True)

    def body(i, carry):
        chunk(2 * i, bufs[0], bufs[1])
        chunk(2 * i + 1, bufs[1], bufs[0])
        return carry
    lax.fori_loop(0, NCHUNK // 2, body, 0)
    pltpu.make_async_copy(ones_v, ds_s.at[srcB], sem_s).wait()
    pltpu.make_async_copy(ones_v, dd_s.at[dstB], sem_s).wait()

    plsc.subcore_barrier()
    pltpu.sync_copy(ds_s.at[pl.ds(off, SL)], degs.at[c, pl.ds(off, SL)])
    pltpu.sync_copy(dd_s.at[pl.ds(off, SL)], degd.at[c, pl.ds(off, SL)])


@functools.partial(
    pl.kernel,
    out_type=jax.ShapeDtypeStruct((NC, 2, NPAD), _f32),
    mesh=_mesh,
    scratch_types=[
        pltpu.VMEM_SHARED((NPAD,), _f32),
        pltpu.VMEM_SHARED((NPAD,), _f32),
        pltpu.VMEM_SHARED((NPAD,), _f32),
        pltpu.VMEM_SHARED((NPAD,), _f32),
        pltpu.VMEM((CHUNK,), jnp.int32),
        pltpu.VMEM((CHUNK,), jnp.int32),
        pltpu.VMEM((CHUNK,), jnp.int32),
        pltpu.VMEM((CHUNK,), jnp.int32),
        pltpu.VMEM((CHUNK,), _f32),
        pltpu.VMEM((CHUNK,), _f32),
        pltpu.VMEM((CHUNK,), _f32),
        pltpu.VMEM((CHUNK,), _f32),
        pltpu.VMEM((SL,), _f32),
        pltpu.SemaphoreType.DMA,
        pltpu.SemaphoreType.DMA,
        pltpu.SemaphoreType.DMA,
    ],
)
def _sc_pass(src_h, dst_h, tbl, out, t0_s, t1_s, a0_s, a1_s, srcA, dstA, srcB, dstB,
             v0A, v1A, v0B, v1B, zb, sem_i, sem_g, sem_s):
    c = lax.axis_index("c")
    s = lax.axis_index("s")
    off = s * SL
    _fill(zb, SL, 0.0)
    pltpu.sync_copy(tbl.at[0, pl.ds(off, SL)], t0_s.at[pl.ds(off, SL)])
    pltpu.sync_copy(tbl.at[1, pl.ds(off, SL)], t1_s.at[pl.ds(off, SL)])
    pltpu.sync_copy(zb, a0_s.at[pl.ds(off, SL)])
    pltpu.sync_copy(zb, a1_s.at[pl.ds(off, SL)])
    plsc.subcore_barrier()

    base = (c * NS + s) * EPT
    bufs = ((srcA, dstA, v0A, v1A), (srcB, dstB, v0B, v1B))
    pltpu.async_copy(src_h.at[pl.ds(base, CHUNK)], srcA, sem_i)
    pltpu.async_copy(dst_h.at[pl.ds(base, CHUNK)], dstA, sem_i)

    def chunk(k, cur, other):
        srcb, dstb, v0, v1 = cur
        o = base + k * CHUNK
        pltpu.make_async_copy(src_h.at[pl.ds(o, CHUNK)], srcb, sem_i).wait()
        pltpu.make_async_copy(dst_h.at[pl.ds(o, CHUNK)], dstb, sem_i).wait()

        g0 = pltpu.async_copy(t0_s.at[srcb], v0, sem_g)
        g1 = pltpu.async_copy(t1_s.at[srcb], v1, sem_g)

        @pl.when(k >= 1)
        def _():
            pltpu.make_async_copy(other[2], a0_s.at[other[1]], sem_s).wait()
            pltpu.make_async_copy(other[3], a1_s.at[other[1]], sem_s).wait()

        @pl.when(k < NCHUNK - 1)
        def _():
            o2 = base + (k + 1) * CHUNK
            pltpu.async_copy(src_h.at[pl.ds(o2, CHUNK)], other[0], sem_i)
            pltpu.async_copy(dst_h.at[pl.ds(o2, CHUNK)], other[1], sem_i)

        g0.wait()
        g1.wait()
        pltpu.async_copy(v0, a0_s.at[dstb], sem_s, add=True)
        pltpu.async_copy(v1, a1_s.at[dstb], sem_s, add=True)

    def body(i, carry):
        chunk(2 * i, bufs[0], bufs[1])
        chunk(2 * i + 1, bufs[1], bufs[0])
        return carry
    lax.fori_loop(0, NCHUNK // 2, body, 0)
    pltpu.make_async_copy(v0B, a0_s.at[dstB], sem_s).wait()
    pltpu.make_async_copy(v1B, a1_s.at[dstB], sem_s).wait()

    plsc.subcore_barrier()
    pltpu.sync_copy(a0_s.at[pl.ds(off, SL)], out.at[c, 0, pl.ds(off, SL)])
    pltpu.sync_copy(a1_s.at[pl.ds(off, SL)], out.at[c, 1, pl.ds(off, SL)])


def _tc_prep_body(degs_ref, degd_ref, feat_ref, t2_ref, ns_ref, nd_ref):
    ds = degs_ref[0:1, :] + degs_ref[1:2, :]
    dd = degd_ref[0:1, :] + degd_ref[1:2, :]
    ns = lax.rsqrt(jnp.maximum(ds, 1.0))
    nd = lax.rsqrt(jnp.maximum(dd, 1.0))
    ns_ref[...] = ns
    nd_ref[...] = nd
    t2_ref[...] = feat_ref[...] * ns


def _tc_dense_body(acc_ref, ns_ref, nd_ref, w2t_ref, b2_ref, w3t_ref, y3_ref):
    a = (acc_ref[0] + acc_ref[1]) * nd_ref[...]
    x2 = jnp.dot(w2t_ref[...], a, preferred_element_type=_f32) + b2_ref[...]
    x2 = jnp.maximum(x2, 0.0)
    y3 = jnp.dot(w3t_ref[...], x2, preferred_element_type=_f32)
    y3_ref[...] = y3 * ns_ref[...]


def _tc_final_body(acc_ref, nd_ref, b3_ref, out_ref):
    out_ref[...] = (acc_ref[0] + acc_ref[1]) * nd_ref[...] + b3_ref[...]


def kernel(features, edge_index, theta_w, theta_b, phi_w, phi_b, w2, b2, w3, b3):
    featT = jnp.pad(features.T, ((0, 0), (0, NPAD - N)))

    src_e = edge_index[0]
    dst_e = edge_index[1]
    degs, degd = _sc_degrees(src_e, dst_e)

    t2, ns, nd = pl.pallas_call(
        _tc_prep_body,
        out_shape=(
            jax.ShapeDtypeStruct((2, NPAD), _f32),
            jax.ShapeDtypeStruct((1, NPAD), _f32),
            jax.ShapeDtypeStruct((1, NPAD), _f32),
        ),
    )(degs, degd, featT)

    acc2 = _sc_pass(src_e, dst_e, t2)

    y3 = pl.pallas_call(
        _tc_dense_body,
        out_shape=jax.ShapeDtypeStruct((2, NPAD), _f32),
    )(acc2, ns, nd, w2.T, b2.reshape(8, 1), w3.T)

    acc3 = _sc_pass(src_e, dst_e, y3)

    outT = pl.pallas_call(
        _tc_final_body,
        out_shape=jax.ShapeDtypeStruct((2, NPAD), _f32),
    )(acc3, nd, b3.reshape(2, 1))

    return outT[:, :N].T


# speedup vs baseline: 121.8752x; 1.0667x over previous
import functools

import jax
import jax.numpy as jnp
from jax import lax
from jax.experimental import pallas as pl
from jax.experimental.pallas import tpu as pltpu
from jax.experimental.pallas import tpu_sc as plsc

N = 100000
NPAD = 100352
E = 6400000
NC = 2
NS = 16
NW = NC * NS
EPT = E // NW
CHUNK = 4000
NCHUNK = EPT // CHUNK
SL = NPAD // NS

_mesh = plsc.VectorSubcoreMesh(core_axis_name="c", subcore_axis_name="s")
_f32 = jnp.float32


def _fill(ref, n, value):
    def body(i, carry):
        ref[pl.ds(i * 16, 16)] = jnp.full((16,), value, _f32)
        return carry
    lax.fori_loop(0, n // 16, body, 0)


@functools.partial(
    pl.kernel,
    out_type=(
        jax.ShapeDtypeStruct((NC, NPAD), _f32),
        jax.ShapeDtypeStruct((NC, NPAD), _f32),
    ),
    mesh=_mesh,
    scratch_types=[
        pltpu.VMEM_SHARED((NPAD,), _f32),
        pltpu.VMEM_SHARED((NPAD,), _f32),
        pltpu.VMEM((CHUNK,), jnp.int32),
        pltpu.VMEM((CHUNK,), jnp.int32),
        pltpu.VMEM((CHUNK,), jnp.int32),
        pltpu.VMEM((CHUNK,), jnp.int32),
        pltpu.VMEM((CHUNK,), _f32),
        pltpu.VMEM((SL,), _f32),
        pltpu.SemaphoreType.DMA,
        pltpu.SemaphoreType.DMA,
    ],
)
def _sc_degrees(src_h, dst_h, degs, degd, ds_s, dd_s, srcA, dstA, srcB, dstB, ones_v,
                zb, sem_i, sem_s):
    c = lax.axis_index("c")
    s = lax.axis_index("s")
    off = s * SL
    _fill(zb, SL, 0.0)
    _fill(ones_v, CHUNK, 1.0)
    pltpu.sync_copy(zb, ds_s.at[pl.ds(off, SL)])
    pltpu.sync_copy(zb, dd_s.at[pl.ds(off, SL)])
    plsc.subcore_barrier()

    base = (c * NS + s) * EPT
    bufs = ((srcA, dstA), (srcB, dstB))
    pltpu.async_copy(src_h.at[pl.ds(base, CHUNK)], srcA, sem_i)
    pltpu.async_copy(dst_h.at[pl.ds(base, CHUNK)], dstA, sem_i)

    def chunk(k, cur, other):
        srcb, dstb = cur
        o = base + k * CHUNK
        pltpu.make_async_copy(src_h.at[pl.ds(o, CHUNK)], srcb, sem_i).wait()
        pltpu.make_async_copy(dst_h.at[pl.ds(o, CHUNK)], dstb, sem_i).wait()

        @pl.when(k >= 1)
        def _():
            pltpu.make_async_copy(ones_v, ds_s.at[other[0]], sem_s).wait()
            pltpu.make_async_copy(ones_v, dd_s.at[other[1]], sem_s).wait()

        @pl.when(k < NCHUNK - 1)
        def _():
            o2 = base + (k + 1) * CHUNK
            pltpu.async_copy(src_h.at[pl.ds(o2, CHUNK)], other[0], sem_i)
            pltpu.async_copy(dst_h.at[pl.ds(o2, CHUNK)], other[1], sem_i)

        pltpu.async_copy(ones_v, ds_s.at[srcb], sem_s, add=True)
        pltpu.async_copy(ones_v, dd_s.at[dstb], sem_s, add=True)

    def body(i, carry):
        chunk(2 * i, bufs[0], bufs[1])
        chunk(2 * i + 1, bufs[1], bufs[0])
        return carry
    lax.fori_loop(0, NCHUNK // 2, body, 0)
    pltpu.make_async_copy(ones_v, ds_s.at[srcB], sem_s).wait()
    pltpu.make_async_copy(ones_v, dd_s.at[dstB], sem_s).wait()

    plsc.subcore_barrier()
    pltpu.sync_copy(ds_s.at[pl.ds(off, SL)], degs.at[c, pl.ds(off, SL)])
    pltpu.sync_copy(dd_s.at[pl.ds(off, SL)], degd.at[c, pl.ds(off, SL)])


@functools.partial(
    pl.kernel,
    out_type=jax.ShapeDtypeStruct((NC, 2, NPAD), _f32),
    mesh=_mesh,
    scratch_types=[
        pltpu.VMEM_SHARED((NPAD,), _f32),
        pltpu.VMEM_SHARED((NPAD,), _f32),
        pltpu.VMEM_SHARED((NPAD,), _f32),
        pltpu.VMEM_SHARED((NPAD,), _f32),
        pltpu.VMEM((CHUNK,), jnp.int32),
        pltpu.VMEM((CHUNK,), jnp.int32),
        pltpu.VMEM((CHUNK,), jnp.int32),
        pltpu.VMEM((CHUNK,), jnp.int32),
        pltpu.VMEM((CHUNK,), _f32),
        pltpu.VMEM((CHUNK,), _f32),
        pltpu.VMEM((CHUNK,), _f32),
        pltpu.VMEM((CHUNK,), _f32),
        pltpu.VMEM((SL,), _f32),
        pltpu.SemaphoreType.DMA,
        pltpu.SemaphoreType.DMA,
        pltpu.SemaphoreType.DMA,
    ],
)
def _sc_pass(src_h, dst_h, tbl, out, t0_s, t1_s, a0_s, a1_s, srcA, dstA, srcB, dstB,
             v0A, v1A, v0B, v1B, zb, sem_i, sem_g, sem_s):
    c = lax.axis_index("c")
    s = lax.axis_index("s")
    off = s * SL
    _fill(zb, SL, 0.0)
    pltpu.sync_copy(tbl.at[0, pl.ds(off, SL)], t0_s.at[pl.ds(off, SL)])
    pltpu.sync_copy(tbl.at[1, pl.ds(off, SL)], t1_s.at[pl.ds(off, SL)])
    pltpu.sync_copy(zb, a0_s.at[pl.ds(off, SL)])
    pltpu.sync_copy(zb, a1_s.at[pl.ds(off, SL)])
    plsc.subcore_barrier()

    base = (c * NS + s) * EPT
    bufs = ((srcA, dstA, v0A, v1A), (srcB, dstB, v0B, v1B))
    pltpu.async_copy(src_h.at[pl.ds(base, CHUNK)], srcA, sem_i)
    pltpu.async_copy(dst_h.at[pl.ds(base, CHUNK)], dstA, sem_i)

    def chunk(k, cur, other):
        srcb, dstb, v0, v1 = cur
        o = base + k * CHUNK
        pltpu.make_async_copy(src_h.at[pl.ds(o, CHUNK)], srcb, sem_i).wait()
        pltpu.make_async_copy(dst_h.at[pl.ds(o, CHUNK)], dstb, sem_i).wait()

        g0 = pltpu.async_copy(t0_s.at[srcb], v0, sem_g)
        g1 = pltpu.async_copy(t1_s.at[srcb], v1, sem_g)

        @pl.when(k >= 1)
        def _():
            pltpu.make_async_copy(other[2], a0_s.at[other[1]], sem_s).wait()
            pltpu.make_async_copy(other[3], a1_s.at[other[1]], sem_s).wait()

        @pl.when(k < NCHUNK - 1)
        def _():
            o2 = base + (k + 1) * CHUNK
            pltpu.async_copy(src_h.at[pl.ds(o2, CHUNK)], other[0], sem_i)
            pltpu.async_copy(dst_h.at[pl.ds(o2, CHUNK)], other[1], sem_i)

        g0.wait()
        g1.wait()
        pltpu.async_copy(v0, a0_s.at[dstb], sem_s, add=True)
        pltpu.async_copy(v1, a1_s.at[dstb], sem_s, add=True)

    def body(i, carry):
        chunk(2 * i, bufs[0], bufs[1])
        chunk(2 * i + 1, bufs[1], bufs[0])
        return carry
    lax.fori_loop(0, NCHUNK // 2, body, 0)
    pltpu.make_async_copy(v0B, a0_s.at[dstB], sem_s).wait()
    pltpu.make_async_copy(v1B, a1_s.at[dstB], sem_s).wait()

    plsc.subcore_barrier()
    pltpu.sync_copy(a0_s.at[pl.ds(off, SL)], out.at[c, 0, pl.ds(off, SL)])
    pltpu.sync_copy(a1_s.at[pl.ds(off, SL)], out.at[c, 1, pl.ds(off, SL)])


def _tc_prep_body(degs_ref, degd_ref, feat_ref, t2_ref, ns_ref, nd_ref):
    ds = degs_ref[0:1, :] + degs_ref[1:2, :]
    dd = degd_ref[0:1, :] + degd_ref[1:2, :]
    ns = lax.rsqrt(jnp.maximum(ds, 1.0))
    nd = lax.rsqrt(jnp.maximum(dd, 1.0))
    ns_ref[...] = ns
    nd_ref[...] = nd
    t2_ref[...] = feat_ref[...] * ns


def _tc_dense_body(acc_ref, ns_ref, nd_ref, w2t_ref, b2_ref, w3t_ref, y3_ref):
    a = (acc_ref[0] + acc_ref[1]) * nd_ref[...]
    x2 = jnp.dot(w2t_ref[...], a, preferred_element_type=_f32) + b2_ref[...]
    x2 = jnp.maximum(x2, 0.0)
    y3 = jnp.dot(w3t_ref[...], x2, preferred_element_type=_f32)
    y3_ref[...] = y3 * ns_ref[...]


def _tc_final_body(acc_ref, nd_ref, b3_ref, out_ref):
    out_ref[...] = (acc_ref[0] + acc_ref[1]) * nd_ref[...] + b3_ref[...]


def kernel(features, edge_index, theta_w, theta_b, phi_w, phi_b, w2, b2, w3, b3):
    featT = jnp.pad(features.T, ((0, 0), (0, NPAD - N)))

    src_e = edge_index[0]
    dst_e = edge_index[1]
    degs, degd = _sc_degrees(src_e, dst_e)

    t2, ns, nd = pl.pallas_call(
        _tc_prep_body,
        out_shape=(
            jax.ShapeDtypeStruct((2, NPAD), _f32),
            jax.ShapeDtypeStruct((1, NPAD), _f32),
            jax.ShapeDtypeStruct((1, NPAD), _f32),
        ),
    )(degs, degd, featT)

    acc2 = _sc_pass(src_e, dst_e, t2)

    y3 = pl.pallas_call(
        _tc_dense_body,
        out_shape=jax.ShapeDtypeStruct((2, NPAD), _f32),
    )(acc2, ns, nd, w2.T, b2.reshape(8, 1), w3.T)

    acc3 = _sc_pass(src_e, dst_e, y3)

    outT = pl.pallas_call(
        _tc_final_body,
        out_shape=jax.ShapeDtypeStruct((2, NPAD), _f32),
    )(acc3, nd, b3.reshape(2, 1))

    return outT[:, :N].T
